# baseline (speedup 1.0000x reference)
import jax
import jax.numpy as jnp
from jax.experimental import pallas as pl
from jax.experimental.pallas import tpu as pltpu

_ROWS = 4000


def _lsh_block(x_ref, rv_ref, out_ref):
    out_ref[...] = x_ref[: _ROWS // 2, :] + rv_ref[0, 0]


def kernel(x, random_vectors):
    n, d = x.shape
    h = random_vectors.shape[1]
    grid = (n // _ROWS,)
    wide = pl.pallas_call(
        _lsh_block,
        grid=grid,
        in_specs=[
            pl.BlockSpec((_ROWS, d), lambda i: (i, 0)),
            pl.BlockSpec((d, h), lambda i: (0, 0)),
        ],
        out_specs=pl.BlockSpec((_ROWS // 2, 2 * h), lambda i: (i, 0)),
        out_shape=jax.ShapeDtypeStruct((n // 2, 2 * h), jnp.float32),
        compiler_params=pltpu.CompilerParams(
            dimension_semantics=("arbitrary",),
        ),
    )(x, random_vectors)
    a = jax.lax.slice(wide, (0, 0), (n // 2, h))
    b = jax.lax.slice(wide, (0, h), (n // 2, 2 * h))
    return jnp.stack([a, b], axis=1).reshape(n, h)


# speedup vs baseline: 2.2397x; 2.2397x over previous
import jax
import jax.numpy as jnp
from jax.experimental import pallas as pl
from jax.experimental.pallas import tpu as pltpu

_ROWS = 10000


def _lsh_block(x_ref, rv_ref, out_ref):
    proj = jnp.dot(x_ref[...], rv_ref[...], preferred_element_type=jnp.float32)
    i = jnp.floor(proj).astype(jnp.int32)
    out_ref[...] = (i & 1023).astype(jnp.float32)


def kernel(x, random_vectors):
    n, d = x.shape
    h = random_vectors.shape[1]
    grid = (n // _ROWS,)
    return pl.pallas_call(
        _lsh_block,
        grid=grid,
        in_specs=[
            pl.BlockSpec((_ROWS, d), lambda i: (i, 0)),
            pl.BlockSpec((d, h), lambda i: (0, 0)),
        ],
        out_specs=pl.BlockSpec((_ROWS, h), lambda i: (i, 0)),
        out_shape=jax.ShapeDtypeStruct((n, h), jnp.float32),
    )(x, random_vectors)


# speedup vs baseline: 2.2688x; 1.0130x over previous
import jax
import jax.numpy as jnp
from jax.experimental import pallas as pl
from jax.experimental.pallas import tpu as pltpu

_ROWS = 20000


def _lsh_block(x_ref, rv_ref, out_ref):
    proj = jnp.dot(x_ref[...], rv_ref[...], preferred_element_type=jnp.float32)
    i = jnp.floor(proj).astype(jnp.int32)
    out_ref[...] = (i & 1023).astype(jnp.float32)


def kernel(x, random_vectors):
    n, d = x.shape
    h = random_vectors.shape[1]
    grid = (n // _ROWS,)
    return pl.pallas_call(
        _lsh_block,
        grid=grid,
        in_specs=[
            pl.BlockSpec((_ROWS, d), lambda i: (i, 0)),
            pl.BlockSpec((d, h), lambda i: (0, 0)),
        ],
        out_specs=pl.BlockSpec((_ROWS, h), lambda i: (i, 0)),
        out_shape=jax.ShapeDtypeStruct((n, h), jnp.float32),
    )(x, random_vectors)
